# baseline (speedup 1.0000x reference)
import functools

import jax
import jax.numpy as jnp
from jax import lax
from jax.experimental import pallas as pl
from jax.experimental.pallas import tpu as pltpu
from jax.experimental.pallas import tpu_sc as plsc

_B = 16384
_D = 64
_NC = 2
_NS = 16
_L = 16
_NW = _NC * _NS
_BPW = _B // _NW


def _dot_sc(u, v, emb, ctx):
    mesh = plsc.VectorSubcoreMesh(core_axis_name="c", subcore_axis_name="s")

    @functools.partial(
        pl.kernel,
        mesh=mesh,
        compiler_params=pltpu.CompilerParams(
            needs_layout_passes=False, use_tc_tiling_on_sc=False
        ),
        out_type=jax.ShapeDtypeStruct((_B,), jnp.float32),
        scratch_types=[
            pltpu.VMEM((_BPW,), jnp.int32),
            pltpu.VMEM((_BPW,), jnp.int32),
            pltpu.VMEM((_BPW, _D), jnp.float32),
            pltpu.VMEM((_BPW, _D), jnp.float32),
            pltpu.VMEM((_BPW,), jnp.float32),
            pltpu.VMEM((_L * _L,), jnp.float32),
            pltpu.SemaphoreType.DMA,
        ],
    )
    def k(u_hbm, v_hbm, emb_hbm, ctx_hbm, out_hbm, ui, vi, xr, yr, dv, tb, sem):
        wid = lax.axis_index("s") * _NC + lax.axis_index("c")
        base = wid * _BPW
        pltpu.sync_copy(u_hbm.at[pl.ds(base, _BPW)], ui)
        pltpu.sync_copy(v_hbm.at[pl.ds(base, _BPW)], vi)
        cx = pltpu.async_copy(emb_hbm.at[ui], xr, sem)
        cy = pltpu.async_copy(ctx_hbm.at[vi], yr, sem)
        cx.wait()
        cy.wait()

        tcols = lax.iota(jnp.int32, _L) * _L

        def grp(g, carry):
            for j in range(_L):
                r = g * _L + j
                t = xr[r, pl.ds(0, _L)] * yr[r, pl.ds(0, _L)]
                for c in range(1, _D // _L):
                    t = t + xr[r, pl.ds(c * _L, _L)] * yr[r, pl.ds(c * _L, _L)]
                plsc.store_scatter(tb, [tcols + j], t)
            acc = tb[pl.ds(0, _L)]
            for i in range(1, _L):
                acc = acc + tb[pl.ds(i * _L, _L)]
            dv[pl.ds(g * _L, _L)] = acc
            return carry

        lax.fori_loop(0, _BPW // _L, grp, 0)
        pltpu.sync_copy(dv, out_hbm.at[pl.ds(base, _BPW)])

    return k(u, v, emb, ctx)


def _loss_tc(d, w):
    def k(d_ref, w_ref, o_ref):
        s = w_ref[...] * d_ref[...]
        ls = jnp.minimum(s, 0.0) - jnp.log1p(jnp.exp(-jnp.abs(s)))
        o_ref[0, 0] = -jnp.sum(ls) * (1.0 / _B)

    out = pl.pallas_call(
        k,
        out_shape=jax.ShapeDtypeStruct((1, 1), jnp.float32),
        out_specs=pl.BlockSpec(memory_space=pltpu.SMEM),
    )(d.reshape(128, 128), w.reshape(128, 128))
    return out[0, 0]


@jax.jit
def kernel(u, v, w, emb, ctx):
    u = u.astype(jnp.int32)
    v = v.astype(jnp.int32)
    d = _dot_sc(u, v, emb, ctx)
    return _loss_tc(d, w.astype(jnp.float32))


# speedup vs baseline: 1.4729x; 1.4729x over previous
import functools

import jax
import jax.numpy as jnp
from jax import lax
from jax.experimental import pallas as pl
from jax.experimental.pallas import tpu as pltpu
from jax.experimental.pallas import tpu_sc as plsc

_B = 16384
_D = 64
_NC = 2
_NS = 16
_L = 16
_NW = _NC * _NS
_BPW = _B // _NW
_C = 32
_NCH = _BPW // _C


def _dot_sc(u, v, emb, ctx):
    mesh = plsc.VectorSubcoreMesh(core_axis_name="c", subcore_axis_name="s")

    @functools.partial(
        pl.kernel,
        mesh=mesh,
        compiler_params=pltpu.CompilerParams(needs_layout_passes=False),
        out_type=jax.ShapeDtypeStruct((_B,), jnp.float32),
        scratch_types=[
            pltpu.VMEM((_BPW,), jnp.int32),
            pltpu.VMEM((_BPW,), jnp.int32),
            pltpu.VMEM((_C, 8, _D), jnp.float32),
            pltpu.VMEM((_C, 8, _D), jnp.float32),
            pltpu.VMEM((_BPW,), jnp.float32),
            pltpu.VMEM((_L * _L,), jnp.float32),
            pltpu.SemaphoreType.DMA,
            pltpu.SemaphoreType.DMA,
        ],
    )
    def k(u_hbm, v_hbm, emb_hbm, ctx_hbm, out_hbm, ui, vi, xb, yb, dv, tb, sx, sy):
        wid = lax.axis_index("s") * _NC + lax.axis_index("c")
        base = wid * _BPW
        pltpu.sync_copy(u_hbm.at[pl.ds(base, _BPW)], ui)
        pltpu.sync_copy(v_hbm.at[pl.ds(base, _BPW)], vi)

        iota = lax.iota(jnp.int32, _L)
        tcols = iota * _L

        def chunk(ch, carry):
            for g in range(_C // _L):
                iu = ui[pl.ds(ch * _C + g * _L, _L)]
                iv = vi[pl.ds(ch * _C + g * _L, _L)]
                tu = iu - (iu & 7)
                tv = iv - (iv & 7)
                for j in range(_L):
                    i = g * _L + j
                    pltpu.async_copy(
                        emb_hbm.at[pl.ds(pl.multiple_of(tu[j], 8), 8)], xb.at[i], sx
                    )
                    pltpu.async_copy(
                        ctx_hbm.at[pl.ds(pl.multiple_of(tv[j], 8), 8)], yb.at[i], sy
                    )
            for g in range(_C // _L):
                for j in range(_L):
                    i = g * _L + j
                    pltpu.make_async_copy(emb_hbm.at[pl.ds(0, 8)], xb.at[i], sx).wait()
                    pltpu.make_async_copy(ctx_hbm.at[pl.ds(0, 8)], yb.at[i], sy).wait()

            for g in range(_C // _L):
                iu = ui[pl.ds(ch * _C + g * _L, _L)]
                iv = vi[pl.ds(ch * _C + g * _L, _L)]
                su = iu & 7
                sv = iv & 7
                for j in range(_L):
                    i = g * _L + j
                    bi = jnp.full((_L,), i, jnp.int32)
                    bsu = jnp.full((_L,), su[j], jnp.int32)
                    bsv = jnp.full((_L,), sv[j], jnp.int32)
                    acc = jnp.zeros((_L,), jnp.float32)
                    for c in range(_D // _L):
                        cols = iota + c * _L
                        xvv = plsc.load_gather(xb, [bi, bsu, cols])
                        yvv = plsc.load_gather(yb, [bi, bsv, cols])
                        acc = acc + xvv * yvv
                    plsc.store_scatter(tb, [tcols + j], acc)
                accv = tb[pl.ds(0, _L)]
                for t in range(1, _L):
                    accv = accv + tb[pl.ds(t * _L, _L)]
                dv[pl.ds(ch * _C + g * _L, _L)] = accv
            return carry

        lax.fori_loop(0, _NCH, chunk, 0)
        pltpu.sync_copy(dv, out_hbm.at[pl.ds(base, _BPW)])

    return k(u, v, emb, ctx)


def _loss_tc(d, w):
    def k(d_ref, w_ref, o_ref):
        s = w_ref[...] * d_ref[...]
        ls = jnp.minimum(s, 0.0) - jnp.log1p(jnp.exp(-jnp.abs(s)))
        o_ref[0, 0] = -jnp.sum(ls) * (1.0 / _B)

    out = pl.pallas_call(
        k,
        out_shape=jax.ShapeDtypeStruct((1, 1), jnp.float32),
        out_specs=pl.BlockSpec(memory_space=pltpu.SMEM),
    )(d.reshape(128, 128), w.reshape(128, 128))
    return out[0, 0]


@jax.jit
def kernel(u, v, w, emb, ctx):
    u = u.astype(jnp.int32)
    v = v.astype(jnp.int32)
    d = _dot_sc(u, v, emb, ctx)
    return _loss_tc(d, w.astype(jnp.float32))


# speedup vs baseline: 1.4736x; 1.0004x over previous
import functools

import jax
import jax.numpy as jnp
from jax import lax
from jax.experimental import pallas as pl
from jax.experimental.pallas import tpu as pltpu
from jax.experimental.pallas import tpu_sc as plsc

_B = 16384
_D = 64
_NC = 2
_NS = 16
_L = 16
_NW = _NC * _NS
_BPW = _B // _NW
_C = 32
_NCH = _BPW // _C


def _dot_sc(u, v, emb, ctx):
    mesh = plsc.VectorSubcoreMesh(core_axis_name="c", subcore_axis_name="s")

    @functools.partial(
        pl.kernel,
        mesh=mesh,
        compiler_params=pltpu.CompilerParams(
            needs_layout_passes=False,
            skip_device_barrier=True,
            disable_bounds_checks=True,
            disable_semaphore_checks=True,
        ),
        out_type=jax.ShapeDtypeStruct((_B,), jnp.float32),
        scratch_types=[
            pltpu.VMEM((_BPW,), jnp.int32),
            pltpu.VMEM((_BPW,), jnp.int32),
            pltpu.VMEM((_C, 8, _D), jnp.float32),
            pltpu.VMEM((_C, 8, _D), jnp.float32),
            pltpu.VMEM((_BPW,), jnp.float32),
            pltpu.VMEM((_L * _L,), jnp.float32),
            pltpu.SemaphoreType.DMA,
            pltpu.SemaphoreType.DMA,
        ],
    )
    def k(u_hbm, v_hbm, emb_hbm, ctx_hbm, out_hbm, ui, vi, xb, yb, dv, tb, sx, sy):
        wid = lax.axis_index("s") * _NC + lax.axis_index("c")
        base = wid * _BPW
        pltpu.sync_copy(u_hbm.at[pl.ds(base, _BPW)], ui)
        pltpu.sync_copy(v_hbm.at[pl.ds(base, _BPW)], vi)

        iota = lax.iota(jnp.int32, _L)
        tcols = iota * _L

        def chunk(ch, carry):
            for g in range(_C // _L):
                iu = ui[pl.ds(ch * _C + g * _L, _L)]
                iv = vi[pl.ds(ch * _C + g * _L, _L)]
                tu = iu - (iu & 7)
                tv = iv - (iv & 7)
                for j in range(_L):
                    i = g * _L + j
                    pltpu.async_copy(
                        emb_hbm.at[pl.ds(pl.multiple_of(tu[j], 8), 8)], xb.at[i], sx
                    )
                    pltpu.async_copy(
                        ctx_hbm.at[pl.ds(pl.multiple_of(tv[j], 8), 8)], yb.at[i], sy
                    )
            for g in range(_C // _L):
                for j in range(_L):
                    i = g * _L + j
                    pltpu.make_async_copy(emb_hbm.at[pl.ds(0, 8)], xb.at[i], sx).wait()
                    pltpu.make_async_copy(ctx_hbm.at[pl.ds(0, 8)], yb.at[i], sy).wait()

            for g in range(_C // _L):
                iu = ui[pl.ds(ch * _C + g * _L, _L)]
                iv = vi[pl.ds(ch * _C + g * _L, _L)]
                su = iu & 7
                sv = iv & 7
                for j in range(_L):
                    i = g * _L + j
                    bi = jnp.full((_L,), i, jnp.int32)
                    bsu = jnp.full((_L,), su[j], jnp.int32)
                    bsv = jnp.full((_L,), sv[j], jnp.int32)
                    acc = jnp.zeros((_L,), jnp.float32)
                    for c in range(_D // _L):
                        cols = iota + c * _L
                        xvv = plsc.load_gather(xb, [bi, bsu, cols])
                        yvv = plsc.load_gather(yb, [bi, bsv, cols])
                        acc = acc + xvv * yvv
                    plsc.store_scatter(tb, [tcols + j], acc)
                accv = tb[pl.ds(0, _L)]
                for t in range(1, _L):
                    accv = accv + tb[pl.ds(t * _L, _L)]
                dv[pl.ds(ch * _C + g * _L, _L)] = accv
            return carry

        lax.fori_loop(0, _NCH, chunk, 0)
        pltpu.sync_copy(dv, out_hbm.at[pl.ds(base, _BPW)])

    return k(u, v, emb, ctx)


def _loss_tc(d, w):
    def k(d_ref, w_ref, o_ref):
        s = w_ref[...] * d_ref[...]
        ls = jnp.minimum(s, 0.0) - jnp.log1p(jnp.exp(-jnp.abs(s)))
        o_ref[0, 0] = -jnp.sum(ls) * (1.0 / _B)

    out = pl.pallas_call(
        k,
        out_shape=jax.ShapeDtypeStruct((1, 1), jnp.float32),
        out_specs=pl.BlockSpec(memory_space=pltpu.SMEM),
    )(d.reshape(128, 128), w.reshape(128, 128))
    return out[0, 0]


@jax.jit
def kernel(u, v, w, emb, ctx):
    u = u.astype(jnp.int32)
    v = v.astype(jnp.int32)
    d = _dot_sc(u, v, emb, ctx)
    return _loss_tc(d, w.astype(jnp.float32))


# speedup vs baseline: 2.1302x; 1.4456x over previous
import functools

import jax
import jax.numpy as jnp
from jax import lax
from jax.experimental import pallas as pl
from jax.experimental.pallas import tpu as pltpu
from jax.experimental.pallas import tpu_sc as plsc

_N = 1000000
_B = 16384
_D = 64
_NC = 2
_NS = 16
_L = 16
_NW = _NC * _NS
_BPW = _B // _NW
_C = 4
_TAIL = (_N // 128) * 128


def _dot_sc(u, v, embT, ctxT):
    mesh = plsc.VectorSubcoreMesh(core_axis_name="c", subcore_axis_name="s")

    @functools.partial(
        pl.kernel,
        mesh=mesh,
        compiler_params=pltpu.CompilerParams(
            needs_layout_passes=False,
            skip_device_barrier=True,
            disable_bounds_checks=True,
            disable_semaphore_checks=True,
        ),
        out_type=jax.ShapeDtypeStruct((_B,), jnp.float32),
        scratch_types=[
            pltpu.VMEM((_BPW,), jnp.int32),
            pltpu.VMEM((_BPW,), jnp.int32),
            pltpu.VMEM((_C, _D, 128), jnp.float32),
            pltpu.VMEM((_C, _D, 128), jnp.float32),
            pltpu.VMEM((_D, _N - _TAIL), jnp.float32),
            pltpu.VMEM((_D, _N - _TAIL), jnp.float32),
            pltpu.VMEM((_BPW,), jnp.float32),
            pltpu.VMEM((_L * _L,), jnp.float32),
            pltpu.SemaphoreType.DMA,
            pltpu.SemaphoreType.DMA,
        ],
    )
    def k(
        u_hbm, v_hbm, embT_hbm, ctxT_hbm, out_hbm,
        ui, vi, xb, yb, xt, yt, dv, tb, sx, sy,
    ):
        wid = lax.axis_index("s") * _NC + lax.axis_index("c")
        base = wid * _BPW
        pltpu.sync_copy(u_hbm.at[pl.ds(base, _BPW)], ui)
        pltpu.sync_copy(v_hbm.at[pl.ds(base, _BPW)], vi)

        iota = lax.iota(jnp.int32, _L)
        tcols = iota * _L

        def fetch(table_hbm, buf, tail, slot, idx, sem):
            blk = pl.multiple_of(idx - (idx & 127), 128)

            @pl.when(idx < _TAIL)
            def _():
                pltpu.async_copy(
                    table_hbm.at[:, pl.ds(blk, 128)], buf.at[slot], sem
                )

            @pl.when(idx >= _TAIL)
            def _():
                pltpu.async_copy(
                    table_hbm.at[:, pl.ds(_TAIL, _N - _TAIL)], tail, sem
                )

        def drain(table_hbm, buf, tail, slot, idx, sem):
            @pl.when(idx < _TAIL)
            def _():
                pltpu.make_async_copy(
                    table_hbm.at[:, pl.ds(0, 128)], buf.at[slot], sem
                ).wait()

            @pl.when(idx >= _TAIL)
            def _():
                pltpu.make_async_copy(
                    table_hbm.at[:, pl.ds(_TAIL, _N - _TAIL)], tail, sem
                ).wait()

        def grp(g, carry):
            iu16 = ui[pl.ds(pl.multiple_of(g * _L, _L), _L)]
            iv16 = vi[pl.ds(pl.multiple_of(g * _L, _L), _L)]
            for q in range(_L // _C):
                for j in range(_C):
                    lane = q * _C + j
                    fetch(embT_hbm, xb, xt, j, iu16[lane], sx)
                    fetch(ctxT_hbm, yb, yt, j, iv16[lane], sy)
                for j in range(_C):
                    lane = q * _C + j
                    drain(embT_hbm, xb, xt, j, iu16[lane], sx)
                    drain(ctxT_hbm, yb, yt, j, iv16[lane], sy)
                for j in range(_C):
                    lane = q * _C + j
                    bru = jnp.full((_L,), iu16[lane] & 127, jnp.int32)
                    brv = jnp.full((_L,), iv16[lane] & 127, jnp.int32)
                    bj = jnp.full((_L,), j, jnp.int32)
                    acc = jnp.zeros((_L,), jnp.float32)
                    for c in range(_D // _L):
                        dims = iota + c * _L
                        xg = plsc.load_gather(xb, [bj, dims, bru])
                        yg = plsc.load_gather(yb, [bj, dims, brv])
                        xg2 = plsc.load_gather(xt, [dims, bru])
                        yg2 = plsc.load_gather(yt, [dims, brv])
                        xvv = jnp.where(iu16[lane] < _TAIL, xg, xg2)
                        yvv = jnp.where(iv16[lane] < _TAIL, yg, yg2)
                        acc = acc + xvv * yvv
                    plsc.store_scatter(tb, [tcols + lane], acc)
            accv = tb[pl.ds(0, _L)]
            for t in range(1, _L):
                accv = accv + tb[pl.ds(t * _L, _L)]
            dv[pl.ds(pl.multiple_of(g * _L, _L), _L)] = accv
            return carry

        lax.fori_loop(0, _BPW // _L, grp, 0)
        pltpu.sync_copy(dv, out_hbm.at[pl.ds(base, _BPW)])

    return k(u, v, embT, ctxT)


def _loss_tc(d, w):
    def k(d_ref, w_ref, o_ref):
        s = w_ref[...] * d_ref[...]
        ls = jnp.minimum(s, 0.0) - jnp.log1p(jnp.exp(-jnp.abs(s)))
        o_ref[0, 0] = -jnp.sum(ls) * (1.0 / _B)

    out = pl.pallas_call(
        k,
        out_shape=jax.ShapeDtypeStruct((1, 1), jnp.float32),
        out_specs=pl.BlockSpec(memory_space=pltpu.SMEM),
    )(d.reshape(128, 128), w.reshape(128, 128))
    return out[0, 0]


@jax.jit
def kernel(u, v, w, emb, ctx):
    u = u.astype(jnp.int32)
    v = v.astype(jnp.int32)
    d = _dot_sc(u, v, emb.T, ctx.T)
    return _loss_tc(d, w.astype(jnp.float32))


# speedup vs baseline: 2.4678x; 1.1585x over previous
import functools

import jax
import jax.numpy as jnp
from jax import lax
from jax.experimental import pallas as pl
from jax.experimental.pallas import tpu as pltpu
from jax.experimental.pallas import tpu_sc as plsc

_N = 1000000
_B = 16384
_D = 64
_NC = 2
_NS = 16
_L = 16
_NW = _NC * _NS
_BPW = _B // _NW
_C = 2
_TAIL = (_N // 128) * 128


def _dot_sc(u, v, embT, ctxT):
    mesh = plsc.VectorSubcoreMesh(core_axis_name="c", subcore_axis_name="s")

    @functools.partial(
        pl.kernel,
        mesh=mesh,
        compiler_params=pltpu.CompilerParams(
            needs_layout_passes=False,
            skip_device_barrier=True,
            disable_bounds_checks=True,
            disable_semaphore_checks=True,
        ),
        out_type=jax.ShapeDtypeStruct((_B,), jnp.float32),
        scratch_types=[
            pltpu.VMEM((_BPW,), jnp.int32),
            pltpu.VMEM((_BPW,), jnp.int32),
            pltpu.VMEM((2, _C, _D, 128), jnp.float32),
            pltpu.VMEM((2, _C, _D, 128), jnp.float32),
            pltpu.VMEM((_D, _N - _TAIL), jnp.float32),
            pltpu.VMEM((_D, _N - _TAIL), jnp.float32),
            pltpu.VMEM((_BPW,), jnp.float32),
            pltpu.VMEM((_L * _L,), jnp.float32),
            pltpu.SemaphoreType.DMA,
            pltpu.SemaphoreType.DMA,
            pltpu.SemaphoreType.DMA,
            pltpu.SemaphoreType.DMA,
        ],
    )
    def k(
        u_hbm, v_hbm, embT_hbm, ctxT_hbm, out_hbm,
        ui, vi, xb, yb, xt, yt, dv, tb, sx0, sx1, sy0, sy1,
    ):
        wid = lax.axis_index("s") * _NC + lax.axis_index("c")
        base = wid * _BPW
        pltpu.sync_copy(u_hbm.at[pl.ds(base, _BPW)], ui)
        pltpu.sync_copy(v_hbm.at[pl.ds(base, _BPW)], vi)

        iota = lax.iota(jnp.int32, _L)
        tcols = iota * _L

        def fetch(table_hbm, buf, tail, slot, idx, sem):
            blk = pl.multiple_of(idx - (idx & 127), 128)

            @pl.when(idx < _TAIL)
            def _():
                pltpu.async_copy(
                    table_hbm.at[:, pl.ds(blk, 128)], buf.at[slot], sem
                )

            @pl.when(idx >= _TAIL)
            def _():
                pltpu.async_copy(
                    table_hbm.at[:, pl.ds(_TAIL, _N - _TAIL)], tail, sem
                )

        def drain(table_hbm, buf, tail, slot, idx, sem):
            @pl.when(idx < _TAIL)
            def _():
                pltpu.make_async_copy(
                    table_hbm.at[:, pl.ds(0, 128)], buf.at[slot], sem
                ).wait()

            @pl.when(idx >= _TAIL)
            def _():
                pltpu.make_async_copy(
                    table_hbm.at[:, pl.ds(_TAIL, _N - _TAIL)], tail, sem
                ).wait()

        nq = _L // _C

        def fetch_round(ph, iu16, iv16, q):
            for j in range(_C):
                lane = q * _C + j
                fetch(embT_hbm, xb.at[ph], xt, j, iu16[lane], sx1 if ph else sx0)
                fetch(ctxT_hbm, yb.at[ph], yt, j, iv16[lane], sy1 if ph else sy0)

        iu_first = ui[pl.ds(0, _L)]
        iv_first = vi[pl.ds(0, _L)]
        fetch_round(0, iu_first, iv_first, 0)

        def grp(g, carry):
            iu16 = ui[pl.ds(pl.multiple_of(g * _L, _L), _L)]
            iv16 = vi[pl.ds(pl.multiple_of(g * _L, _L), _L)]
            gn = jnp.minimum(g + 1, _BPW // _L - 1)
            iu_next = ui[pl.ds(pl.multiple_of(gn * _L, _L), _L)]
            iv_next = vi[pl.ds(pl.multiple_of(gn * _L, _L), _L)]
            for q in range(nq):
                ph = q & 1
                nph = (q + 1) & 1
                if q + 1 < nq:
                    fetch_round(nph, iu16, iv16, q + 1)
                else:
                    fetch_round(nph, iu_next, iv_next, 0)
                for j in range(_C):
                    lane = q * _C + j
                    drain(embT_hbm, xb.at[ph], xt, j, iu16[lane], sx1 if ph else sx0)
                    drain(ctxT_hbm, yb.at[ph], yt, j, iv16[lane], sy1 if ph else sy0)
                for j in range(_C):
                    lane = q * _C + j
                    bru = jnp.full((_L,), iu16[lane] & 127, jnp.int32)
                    brv = jnp.full((_L,), iv16[lane] & 127, jnp.int32)
                    bj = jnp.full((_L,), j, jnp.int32)
                    bp = jnp.full((_L,), ph, jnp.int32)
                    acc = jnp.zeros((_L,), jnp.float32)
                    for c in range(_D // _L):
                        dims = iota + c * _L
                        xg = plsc.load_gather(xb, [bp, bj, dims, bru])
                        yg = plsc.load_gather(yb, [bp, bj, dims, brv])
                        xg2 = plsc.load_gather(xt, [dims, bru])
                        yg2 = plsc.load_gather(yt, [dims, brv])
                        xvv = jnp.where(iu16[lane] < _TAIL, xg, xg2)
                        yvv = jnp.where(iv16[lane] < _TAIL, yg, yg2)
                        acc = acc + xvv * yvv
                    plsc.store_scatter(tb, [tcols + lane], acc)
            accv = tb[pl.ds(0, _L)]
            for t in range(1, _L):
                accv = accv + tb[pl.ds(t * _L, _L)]
            dv[pl.ds(pl.multiple_of(g * _L, _L), _L)] = accv
            return carry

        lax.fori_loop(0, _BPW // _L, grp, 0)
        iu_last = ui[pl.ds(_BPW - _L, _L)]
        iv_last = vi[pl.ds(_BPW - _L, _L)]
        for j in range(_C):
            drain(embT_hbm, xb.at[0], xt, j, iu_last[j], sx0)
            drain(ctxT_hbm, yb.at[0], yt, j, iv_last[j], sy0)
        pltpu.sync_copy(dv, out_hbm.at[pl.ds(base, _BPW)])

    return k(u, v, embT, ctxT)


def _loss_tc(d, w):
    def k(d_ref, w_ref, o_ref):
        s = w_ref[...] * d_ref[...]
        ls = jnp.minimum(s, 0.0) - jnp.log1p(jnp.exp(-jnp.abs(s)))
        o_ref[0, 0] = -jnp.sum(ls) * (1.0 / _B)

    out = pl.pallas_call(
        k,
        out_shape=jax.ShapeDtypeStruct((1, 1), jnp.float32),
        out_specs=pl.BlockSpec(memory_space=pltpu.SMEM),
    )(d.reshape(128, 128), w.reshape(128, 128))
    return out[0, 0]


@jax.jit
def kernel(u, v, w, emb, ctx):
    u = u.astype(jnp.int32)
    v = v.astype(jnp.int32)
    d = _dot_sc(u, v, emb.T, ctx.T)
    return _loss_tc(d, w.astype(jnp.float32))


# speedup vs baseline: 2.6910x; 1.0904x over previous
import functools

import jax
import jax.numpy as jnp
from jax import lax
from jax.experimental import pallas as pl
from jax.experimental.pallas import tpu as pltpu
from jax.experimental.pallas import tpu_sc as plsc

_N = 1000000
_B = 16384
_D = 64
_NC = 2
_NS = 16
_L = 16
_NW = _NC * _NS
_BPW = _B // _NW
_NBLK = (_N + 127) // 128
_TAIL = (_N // 128) * 128
_BASE = _NBLK // _NW
_REM = _NBLK % _NW
_MAXBLK = _BASE + 1

_SC_PARAMS = pltpu.CompilerParams(
    needs_layout_passes=False,
    skip_device_barrier=True,
    disable_bounds_checks=True,
    disable_semaphore_checks=True,
)


def _gather_sc(u, v, embT, ctxT):
    mesh = plsc.VectorSubcoreMesh(core_axis_name="c", subcore_axis_name="s")

    @functools.partial(
        pl.kernel,
        mesh=mesh,
        compiler_params=_SC_PARAMS,
        out_type=(
            jax.ShapeDtypeStruct((_B * _D,), jnp.float32),
            jax.ShapeDtypeStruct((_B * _D,), jnp.float32),
        ),
        scratch_types=[
            pltpu.VMEM((_B,), jnp.int32),
            pltpu.VMEM((_B,), jnp.int32),
            pltpu.VMEM((_B,), jnp.int32),
            pltpu.VMEM((256,), jnp.int32),
            pltpu.VMEM((256,), jnp.int32),
            pltpu.VMEM((256,), jnp.int32),
            pltpu.VMEM((2, _D, 128), jnp.float32),
            pltpu.VMEM((_D, _N - _TAIL), jnp.float32),
            pltpu.VMEM((8, _D), jnp.float32),
            pltpu.SemaphoreType.DMA,
            pltpu.SemaphoreType.DMA,
        ],
    )
    def k(
        u_hbm, v_hbm, embT_hbm, ctxT_hbm, xg_hbm, yg_hbm,
        idxs, plist, olist, hist, starts, wrk, blk, tbuf, ring, sb, so,
    ):
        wid = lax.axis_index("s") * _NC + lax.axis_index("c")
        lo = wid * _BASE + jnp.minimum(wid, _REM)
        nblk = _BASE + (wid < _REM).astype(jnp.int32)

        iota = lax.iota(jnp.int32, _L)
        ones = jnp.ones((_L,), jnp.int32)

        def fetch(table_hbm, b, ph):
            gb = lo + b

            @pl.when(gb < _NBLK - 1)
            def _():
                pltpu.async_copy(
                    table_hbm.at[:, pl.ds(pl.multiple_of(gb * 128, 128), 128)],
                    blk.at[ph],
                    sb,
                )

            @pl.when(gb == _NBLK - 1)
            def _():
                pltpu.async_copy(
                    table_hbm.at[:, pl.ds(_TAIL, _N - _TAIL)], tbuf, sb
                )

        def drain(table_hbm, b, ph):
            gb = lo + b

            @pl.when(gb < _NBLK - 1)
            def _():
                pltpu.make_async_copy(
                    table_hbm.at[:, pl.ds(0, 128)], blk.at[ph], sb
                ).wait()

            @pl.when(gb == _NBLK - 1)
            def _():
                pltpu.make_async_copy(
                    table_hbm.at[:, pl.ds(_TAIL, _N - _TAIL)], tbuf, sb
                ).wait()

        def phase(idx_hbm, table_hbm, out_hbm):
            pltpu.sync_copy(idx_hbm, idxs)
            for kk in range(256 // _L):
                hist[pl.ds(kk * _L, _L)] = jnp.zeros((_L,), jnp.int32)

            def match(kk, off):
                vals = idxs[pl.ds(pl.multiple_of(kk * _L, _L), _L)]
                blkv = lax.shift_right_logical(vals, 7)
                m = (blkv >= lo) & (blkv < lo + nblk)
                pos = kk * _L + iota
                plsc.store_compressed(plist.at[pl.ds(off, _L)], pos, mask=m)
                brel = jnp.where(m, blkv - lo, 0)
                for j in range(_L):
                    mj = m & (iota == j)
                    plsc.addupdate_scatter(hist, [brel], ones, mask=mj)
                cnt = plsc.all_reduce_population_count(m)
                return off + cnt[0]

            nm = lax.fori_loop(0, _B // _L, match, jnp.int32(0))

            def scan(kk, carry):
                c = hist[pl.ds(pl.multiple_of(kk * _L, _L), _L)]
                cum = plsc.cumsum(c)
                ex = cum - c + carry
                starts[pl.ds(pl.multiple_of(kk * _L, _L), _L)] = ex
                wrk[pl.ds(pl.multiple_of(kk * _L, _L), _L)] = ex
                return carry + cum[_L - 1]

            lax.fori_loop(0, 256 // _L, scan, jnp.int32(0))

            def place(kk, carry):
                pos16 = plist[pl.ds(pl.multiple_of(kk * _L, _L), _L)]
                valid = (kk * _L + iota) < nm
                pos16 = jnp.where(valid, pos16, 0)
                vals = plsc.load_gather(idxs, [pos16])
                brel = jnp.where(
                    valid, lax.shift_right_logical(vals, 7) - lo, 0
                )
                for j in range(_L):
                    mj = valid & (iota == j)
                    slotv = plsc.load_gather(wrk, [brel])
                    plsc.store_scatter(olist, [slotv], pos16, mask=mj)
                    plsc.addupdate_scatter(wrk, [brel], ones, mask=mj)
                return carry

            lax.fori_loop(0, (nm + _L - 1) // _L, place, jnp.int32(0))

            fetch(table_hbm, jnp.int32(0), 0)

            def stream(b, cnt):
                ph = b & 1

                @pl.when(b + 1 < nblk)
                def _():
                    fetch(table_hbm, b + 1, (b + 1) & 1)

                drain(table_hbm, b, ph)
                sv = plsc.load_gather(starts, [jnp.full((_L,), b, jnp.int32)])
                hv = plsc.load_gather(hist, [jnp.full((_L,), b, jnp.int32)])
                st = sv[0]
                en = st + hv[0]
                is_tail = (lo + b) == (_NBLK - 1)
                bp = jnp.full((_L,), ph, jnp.int32)

                def item(it, cnt2):
                    pos = plsc.load_gather(olist, [jnp.full((_L,), it, jnp.int32)])[0]
                    val = plsc.load_gather(idxs, [jnp.full((_L,), pos, jnp.int32)])[0]
                    rs = jnp.full((_L,), val & 127, jnp.int32)
                    slot = cnt2 & 7

                    @pl.when(cnt2 >= 8)
                    def _():
                        pltpu.make_async_copy(
                            table_hbm.at[0, pl.ds(0, _D)], ring.at[slot], so
                        ).wait()

                    for c in range(_D // _L):
                        dims = iota + c * _L
                        g1 = plsc.load_gather(blk, [bp, dims, rs])
                        g2 = plsc.load_gather(tbuf, [dims, rs])
                        ring[slot, pl.ds(c * _L, _L)] = jnp.where(
                            is_tail, g2, g1
                        )
                    pltpu.async_copy(
                        ring.at[slot],
                        out_hbm.at[pl.ds(pl.multiple_of(pos * _D, 8), _D)],
                        so,
                    )
                    return cnt2 + 1

                return lax.fori_loop(st, en, item, cnt)

            cnt = lax.fori_loop(0, nblk, stream, jnp.int32(0))
            for j in range(8):
                @pl.when(j < cnt)
                def _():
                    pltpu.make_async_copy(
                        table_hbm.at[0, pl.ds(0, _D)], ring.at[j], so
                    ).wait()

        phase(u_hbm, embT_hbm, xg_hbm)
        phase(v_hbm, ctxT_hbm, yg_hbm)

    return k(u, v, embT, ctxT)


def _dot_sc(xg, yg):
    mesh = plsc.VectorSubcoreMesh(core_axis_name="c", subcore_axis_name="s")

    @functools.partial(
        pl.kernel,
        mesh=mesh,
        compiler_params=_SC_PARAMS,
        out_type=jax.ShapeDtypeStruct((_B,), jnp.float32),
        scratch_types=[
            pltpu.VMEM((_BPW * _D,), jnp.float32),
            pltpu.VMEM((_BPW * _D,), jnp.float32),
            pltpu.VMEM((_BPW,), jnp.float32),
            pltpu.VMEM((_L * _L,), jnp.float32),
        ],
    )
    def k(xg_hbm, yg_hbm, out_hbm, xv, yv, dv, tb):
        wid = lax.axis_index("s") * _NC + lax.axis_index("c")
        base = wid * _BPW
        pltpu.sync_copy(xg_hbm.at[pl.ds(base * _D, _BPW * _D)], xv)
        pltpu.sync_copy(yg_hbm.at[pl.ds(base * _D, _BPW * _D)], yv)
        iota = lax.iota(jnp.int32, _L)
        tcols = iota * _L

        def grp(g, carry):
            for j in range(_L):
                off = (g * _L + j) * _D
                t = xv[pl.ds(off, _L)] * yv[pl.ds(off, _L)]
                for c in range(1, _D // _L):
                    t = t + xv[pl.ds(off + c * _L, _L)] * yv[pl.ds(off + c * _L, _L)]
                plsc.store_scatter(tb, [tcols + j], t)
            accv = tb[pl.ds(0, _L)]
            for t in range(1, _L):
                accv = accv + tb[pl.ds(t * _L, _L)]
            dv[pl.ds(pl.multiple_of(g * _L, _L), _L)] = accv
            return carry

        lax.fori_loop(0, _BPW // _L, grp, 0)
        pltpu.sync_copy(dv, out_hbm.at[pl.ds(base, _BPW)])

    return k(xg, yg)


def _loss_tc(d, w):
    def k(d_ref, w_ref, o_ref):
        s = w_ref[...] * d_ref[...]
        ls = jnp.minimum(s, 0.0) - jnp.log1p(jnp.exp(-jnp.abs(s)))
        o_ref[0, 0] = -jnp.sum(ls) * (1.0 / _B)

    out = pl.pallas_call(
        k,
        out_shape=jax.ShapeDtypeStruct((1, 1), jnp.float32),
        out_specs=pl.BlockSpec(memory_space=pltpu.SMEM),
    )(d.reshape(128, 128), w.reshape(128, 128))
    return out[0, 0]


@jax.jit
def kernel(u, v, w, emb, ctx):
    u = u.astype(jnp.int32)
    v = v.astype(jnp.int32)
    xg, yg = _gather_sc(u, v, emb.T, ctx.T)
    d = _dot_sc(xg, yg)
    return _loss_tc(d, w.astype(jnp.float32))


# speedup vs baseline: 2.8901x; 1.0740x over previous
import functools

import jax
import jax.numpy as jnp
from jax import lax
from jax.experimental import pallas as pl
from jax.experimental.pallas import tpu as pltpu
from jax.experimental.pallas import tpu_sc as plsc

_N = 1000000
_B = 16384
_D = 64
_NC = 2
_NS = 16
_L = 16
_NW = _NC * _NS
_BPW = _B // _NW
_NBLK = (_N + 127) // 128
_TAIL = (_N // 128) * 128
_BASE = _NBLK // _NW
_REM = _NBLK % _NW
_MAXBLK = _BASE + 1

_SC_PARAMS = pltpu.CompilerParams(
    needs_layout_passes=False,
    skip_device_barrier=True,
    disable_bounds_checks=True,
    disable_semaphore_checks=True,
)


def _gather_sc(u, v, embT, ctxT):
    mesh = plsc.VectorSubcoreMesh(core_axis_name="c", subcore_axis_name="s")

    @functools.partial(
        pl.kernel,
        mesh=mesh,
        compiler_params=_SC_PARAMS,
        out_type=(
            jax.ShapeDtypeStruct((_B * _D,), jnp.float32),
            jax.ShapeDtypeStruct((_B * _D,), jnp.float32),
        ),
        scratch_types=[
            pltpu.VMEM((_B,), jnp.int32),
            pltpu.VMEM((_B,), jnp.int32),
            pltpu.VMEM((_B,), jnp.int32),
            pltpu.VMEM((256,), jnp.int32),
            pltpu.VMEM((256,), jnp.int32),
            pltpu.VMEM((256,), jnp.int32),
            pltpu.VMEM((256,), jnp.int32),
            pltpu.VMEM((2, _D, 128), jnp.float32),
            pltpu.VMEM((_D, _N - _TAIL), jnp.float32),
            pltpu.VMEM((8, _D), jnp.float32),
            pltpu.SemaphoreType.DMA,
            pltpu.SemaphoreType.DMA,
        ],
    )
    def k(
        u_hbm, v_hbm, embT_hbm, ctxT_hbm, xg_hbm, yg_hbm,
        idxs, plist, olist, hist, starts, wrk, nlist, blk, tbuf, ring, sb, so,
    ):
        wid = lax.axis_index("s") * _NC + lax.axis_index("c")
        lo = wid * _BASE + jnp.minimum(wid, _REM)
        nblk = _BASE + (wid < _REM).astype(jnp.int32)

        iota = lax.iota(jnp.int32, _L)
        ones = jnp.ones((_L,), jnp.int32)

        def fetch(table_hbm, b, ph):
            gb = lo + b

            @pl.when(gb < _NBLK - 1)
            def _():
                pltpu.async_copy(
                    table_hbm.at[:, pl.ds(pl.multiple_of(gb * 128, 128), 128)],
                    blk.at[ph],
                    sb,
                )

            @pl.when(gb == _NBLK - 1)
            def _():
                pltpu.async_copy(
                    table_hbm.at[:, pl.ds(_TAIL, _N - _TAIL)], tbuf, sb
                )

        def drain(table_hbm, b, ph):
            gb = lo + b

            @pl.when(gb < _NBLK - 1)
            def _():
                pltpu.make_async_copy(
                    table_hbm.at[:, pl.ds(0, 128)], blk.at[ph], sb
                ).wait()

            @pl.when(gb == _NBLK - 1)
            def _():
                pltpu.make_async_copy(
                    table_hbm.at[:, pl.ds(_TAIL, _N - _TAIL)], tbuf, sb
                ).wait()

        def phase(idx_hbm, table_hbm, out_hbm):
            pltpu.sync_copy(idx_hbm, idxs)
            for kk in range(256 // _L):
                hist[pl.ds(kk * _L, _L)] = jnp.zeros((_L,), jnp.int32)

            def match(kk, off):
                vals = idxs[pl.ds(pl.multiple_of(kk * _L, _L), _L)]
                blkv = lax.shift_right_logical(vals, 7)
                m = (blkv >= lo) & (blkv < lo + nblk)
                pos = kk * _L + iota
                plsc.store_compressed(plist.at[pl.ds(off, _L)], pos, mask=m)
                brel = jnp.where(m, blkv - lo, 0)
                for j in range(_L):
                    mj = m & (iota == j)
                    plsc.addupdate_scatter(hist, [brel], ones, mask=mj)
                cnt = plsc.all_reduce_population_count(m)
                return off + cnt[0]

            nm = lax.fori_loop(0, _B // _L, match, jnp.int32(0))

            def scan(kk, carry):
                c = hist[pl.ds(pl.multiple_of(kk * _L, _L), _L)]
                cum = plsc.cumsum(c)
                ex = cum - c + carry
                starts[pl.ds(pl.multiple_of(kk * _L, _L), _L)] = ex
                wrk[pl.ds(pl.multiple_of(kk * _L, _L), _L)] = ex
                return carry + cum[_L - 1]

            lax.fori_loop(0, 256 // _L, scan, jnp.int32(0))

            def place(kk, carry):
                pos16 = plist[pl.ds(pl.multiple_of(kk * _L, _L), _L)]
                valid = (kk * _L + iota) < nm
                pos16 = jnp.where(valid, pos16, 0)
                vals = plsc.load_gather(idxs, [pos16])
                brel = jnp.where(
                    valid, lax.shift_right_logical(vals, 7) - lo, 0
                )
                for j in range(_L):
                    mj = valid & (iota == j)
                    slotv = plsc.load_gather(wrk, [brel])
                    plsc.store_scatter(olist, [slotv], pos16, mask=mj)
                    plsc.addupdate_scatter(wrk, [brel], ones, mask=mj)
                return carry

            lax.fori_loop(0, (nm + _L - 1) // _L, place, jnp.int32(0))

            def needed(kk, offn):
                ids = kk * _L + iota
                m = hist[pl.ds(pl.multiple_of(kk * _L, _L), _L)] > 0
                plsc.store_compressed(nlist.at[pl.ds(offn, _L)], ids, mask=m)
                cn = plsc.all_reduce_population_count(m)
                return offn + cn[0]

            nn = lax.fori_loop(0, 256 // _L, needed, jnp.int32(0))

            def blk_at(i):
                return plsc.load_gather(nlist, [jnp.full((_L,), i, jnp.int32)])[0]

            @pl.when(nn > 0)
            def _():
                fetch(table_hbm, blk_at(jnp.int32(0)), 0)

            def stream(i, cnt):
                b = blk_at(i)
                ph = i & 1

                @pl.when(i + 1 < nn)
                def _():
                    fetch(table_hbm, blk_at(i + 1), (i + 1) & 1)

                drain(table_hbm, b, ph)
                sv = plsc.load_gather(starts, [jnp.full((_L,), b, jnp.int32)])
                hv = plsc.load_gather(hist, [jnp.full((_L,), b, jnp.int32)])
                st = sv[0]
                en = st + hv[0]
                is_tail = (lo + b) == (_NBLK - 1)
                bp = jnp.full((_L,), ph, jnp.int32)

                def item(it, cnt2):
                    pos = plsc.load_gather(olist, [jnp.full((_L,), it, jnp.int32)])[0]
                    val = plsc.load_gather(idxs, [jnp.full((_L,), pos, jnp.int32)])[0]
                    rs = jnp.full((_L,), val & 127, jnp.int32)
                    slot = cnt2 & 7

                    @pl.when(cnt2 >= 8)
                    def _():
                        pltpu.make_async_copy(
                            table_hbm.at[0, pl.ds(0, _D)], ring.at[slot], so
                        ).wait()

                    for c in range(_D // _L):
                        dims = iota + c * _L
                        g1 = plsc.load_gather(blk, [bp, dims, rs])
                        g2 = plsc.load_gather(tbuf, [dims, rs])
                        ring[slot, pl.ds(c * _L, _L)] = jnp.where(
                            is_tail, g2, g1
                        )
                    pltpu.async_copy(
                        ring.at[slot],
                        out_hbm.at[pl.ds(pl.multiple_of(pos * _D, 8), _D)],
                        so,
                    )
                    return cnt2 + 1

                return lax.fori_loop(st, en, item, cnt)

            cnt = lax.fori_loop(0, nn, stream, jnp.int32(0))
            for j in range(8):
                @pl.when(j < cnt)
                def _():
                    pltpu.make_async_copy(
                        table_hbm.at[0, pl.ds(0, _D)], ring.at[j], so
                    ).wait()

        phase(u_hbm, embT_hbm, xg_hbm)
        phase(v_hbm, ctxT_hbm, yg_hbm)

    return k(u, v, embT, ctxT)


def _dot_sc(xg, yg):
    mesh = plsc.VectorSubcoreMesh(core_axis_name="c", subcore_axis_name="s")

    @functools.partial(
        pl.kernel,
        mesh=mesh,
        compiler_params=_SC_PARAMS,
        out_type=jax.ShapeDtypeStruct((_B,), jnp.float32),
        scratch_types=[
            pltpu.VMEM((_BPW * _D,), jnp.float32),
            pltpu.VMEM((_BPW * _D,), jnp.float32),
            pltpu.VMEM((_BPW,), jnp.float32),
            pltpu.VMEM((_L * _L,), jnp.float32),
        ],
    )
    def k(xg_hbm, yg_hbm, out_hbm, xv, yv, dv, tb):
        wid = lax.axis_index("s") * _NC + lax.axis_index("c")
        base = wid * _BPW
        pltpu.sync_copy(xg_hbm.at[pl.ds(base * _D, _BPW * _D)], xv)
        pltpu.sync_copy(yg_hbm.at[pl.ds(base * _D, _BPW * _D)], yv)
        iota = lax.iota(jnp.int32, _L)
        tcols = iota * _L

        def grp(g, carry):
            for j in range(_L):
                off = (g * _L + j) * _D
                t = xv[pl.ds(off, _L)] * yv[pl.ds(off, _L)]
                for c in range(1, _D // _L):
                    t = t + xv[pl.ds(off + c * _L, _L)] * yv[pl.ds(off + c * _L, _L)]
                plsc.store_scatter(tb, [tcols + j], t)
            accv = tb[pl.ds(0, _L)]
            for t in range(1, _L):
                accv = accv + tb[pl.ds(t * _L, _L)]
            dv[pl.ds(pl.multiple_of(g * _L, _L), _L)] = accv
            return carry

        lax.fori_loop(0, _BPW // _L, grp, 0)
        pltpu.sync_copy(dv, out_hbm.at[pl.ds(base, _BPW)])

    return k(xg, yg)


def _loss_tc(d, w):
    def k(d_ref, w_ref, o_ref):
        s = w_ref[...] * d_ref[...]
        ls = jnp.minimum(s, 0.0) - jnp.log1p(jnp.exp(-jnp.abs(s)))
        o_ref[0, 0] = -jnp.sum(ls) * (1.0 / _B)

    out = pl.pallas_call(
        k,
        out_shape=jax.ShapeDtypeStruct((1, 1), jnp.float32),
        out_specs=pl.BlockSpec(memory_space=pltpu.SMEM),
    )(d.reshape(128, 128), w.reshape(128, 128))
    return out[0, 0]


@jax.jit
def kernel(u, v, w, emb, ctx):
    u = u.astype(jnp.int32)
    v = v.astype(jnp.int32)
    xg, yg = _gather_sc(u, v, emb.T, ctx.T)
    d = _dot_sc(xg, yg)
    return _loss_tc(d, w.astype(jnp.float32))


# speedup vs baseline: 3.0290x; 1.0481x over previous
import functools

import jax
import jax.numpy as jnp
from jax import lax
from jax.experimental import pallas as pl
from jax.experimental.pallas import tpu as pltpu
from jax.experimental.pallas import tpu_sc as plsc

_N = 1000000
_B = 16384
_D = 64
_NC = 2
_NS = 16
_L = 16
_NW = _NC * _NS
_BPW = _B // _NW
_NBLK = (_N + 127) // 128
_TAIL = (_N // 128) * 128
_BASE = _NBLK // _NW
_REM = _NBLK % _NW
_MAXBLK = _BASE + 1

_SC_PARAMS = pltpu.CompilerParams(
    needs_layout_passes=False,
    skip_device_barrier=True,
    disable_bounds_checks=True,
    disable_semaphore_checks=True,
)


def _gather_sc(u, v, embT, ctxT):
    mesh = plsc.VectorSubcoreMesh(core_axis_name="c", subcore_axis_name="s")

    @functools.partial(
        pl.kernel,
        mesh=mesh,
        compiler_params=_SC_PARAMS,
        out_type=(
            jax.ShapeDtypeStruct((_B * _D,), jnp.float32),
            jax.ShapeDtypeStruct((_B * _D,), jnp.float32),
        ),
        scratch_types=[
            pltpu.VMEM((_B,), jnp.int32),
            pltpu.VMEM((_B,), jnp.int32),
            pltpu.VMEM((_B,), jnp.int32),
            pltpu.VMEM((256,), jnp.int32),
            pltpu.VMEM((256,), jnp.int32),
            pltpu.VMEM((256,), jnp.int32),
            pltpu.VMEM((256,), jnp.int32),
            pltpu.VMEM((2, _D, 128), jnp.float32),
            pltpu.VMEM((_D, _N - _TAIL), jnp.float32),
            pltpu.VMEM((8, _D), jnp.float32),
            pltpu.SemaphoreType.DMA,
            pltpu.SemaphoreType.DMA,
        ],
    )
    def k(
        u_hbm, v_hbm, embT_hbm, ctxT_hbm, xg_hbm, yg_hbm,
        idxs, plist, olist, hist, starts, wrk, nlist, blk, tbuf, ring, sb, so,
    ):
        wid = lax.axis_index("s") * _NC + lax.axis_index("c")
        lo = wid * _BASE + jnp.minimum(wid, _REM)
        nblk = _BASE + (wid < _REM).astype(jnp.int32)

        iota = lax.iota(jnp.int32, _L)
        ones = jnp.ones((_L,), jnp.int32)

        def fetch(table_hbm, b, ph):
            gb = lo + b

            @pl.when(gb < _NBLK - 1)
            def _():
                pltpu.async_copy(
                    table_hbm.at[:, pl.ds(pl.multiple_of(gb * 128, 128), 128)],
                    blk.at[ph],
                    sb,
                )

            @pl.when(gb == _NBLK - 1)
            def _():
                pltpu.async_copy(
                    table_hbm.at[:, pl.ds(_TAIL, _N - _TAIL)], tbuf, sb
                )

        def drain(table_hbm, b, ph):
            gb = lo + b

            @pl.when(gb < _NBLK - 1)
            def _():
                pltpu.make_async_copy(
                    table_hbm.at[:, pl.ds(0, 128)], blk.at[ph], sb
                ).wait()

            @pl.when(gb == _NBLK - 1)
            def _():
                pltpu.make_async_copy(
                    table_hbm.at[:, pl.ds(_TAIL, _N - _TAIL)], tbuf, sb
                ).wait()

        def phase(idx_hbm, table_hbm, out_hbm):
            pltpu.sync_copy(idx_hbm, idxs)
            for kk in range(256 // _L):
                hist[pl.ds(kk * _L, _L)] = jnp.zeros((_L,), jnp.int32)

            def match(kk, off):
                vals = idxs[pl.ds(pl.multiple_of(kk * _L, _L), _L)]
                blkv = lax.shift_right_logical(vals, 7)
                m = (blkv >= lo) & (blkv < lo + nblk)
                pos = kk * _L + iota
                plsc.store_compressed(plist.at[pl.ds(off, _L)], pos, mask=m)
                brel = jnp.where(m, blkv - lo, 0)
                for j in range(_L):
                    mj = m & (iota == j)
                    plsc.addupdate_scatter(hist, [brel], ones, mask=mj)
                cnt = plsc.all_reduce_population_count(m)
                return off + cnt[0]

            nm = lax.fori_loop(0, _B // _L, match, jnp.int32(0))

            def scan(kk, carry):
                c = hist[pl.ds(pl.multiple_of(kk * _L, _L), _L)]
                cum = plsc.cumsum(c)
                ex = cum - c + carry
                starts[pl.ds(pl.multiple_of(kk * _L, _L), _L)] = ex
                wrk[pl.ds(pl.multiple_of(kk * _L, _L), _L)] = ex
                return carry + cum[_L - 1]

            lax.fori_loop(0, 256 // _L, scan, jnp.int32(0))

            def place(kk, carry):
                pos16 = plist[pl.ds(pl.multiple_of(kk * _L, _L), _L)]
                valid = (kk * _L + iota) < nm
                pos16 = jnp.where(valid, pos16, 0)
                vals = plsc.load_gather(idxs, [pos16])
                brel = jnp.where(
                    valid, lax.shift_right_logical(vals, 7) - lo, 0
                )
                packed = pos16 | lax.shift_left(vals & 127, 14)
                for j in range(_L):
                    mj = valid & (iota == j)
                    slotv = plsc.load_gather(wrk, [brel])
                    plsc.store_scatter(olist, [slotv], packed, mask=mj)
                    plsc.addupdate_scatter(wrk, [brel], ones, mask=mj)
                return carry

            lax.fori_loop(0, (nm + _L - 1) // _L, place, jnp.int32(0))

            def needed(kk, offn):
                ids = kk * _L + iota
                m = hist[pl.ds(pl.multiple_of(kk * _L, _L), _L)] > 0
                plsc.store_compressed(nlist.at[pl.ds(offn, _L)], ids, mask=m)
                cn = plsc.all_reduce_population_count(m)
                return offn + cn[0]

            nn = lax.fori_loop(0, 256 // _L, needed, jnp.int32(0))

            def blk_at(i):
                return plsc.load_gather(nlist, [jnp.full((_L,), i, jnp.int32)])[0]

            @pl.when(nn > 0)
            def _():
                fetch(table_hbm, blk_at(jnp.int32(0)), 0)

            def stream(i, cnt):
                b = blk_at(i)
                ph = i & 1

                @pl.when(i + 1 < nn)
                def _():
                    fetch(table_hbm, blk_at(i + 1), (i + 1) & 1)

                drain(table_hbm, b, ph)
                sv = plsc.load_gather(starts, [jnp.full((_L,), b, jnp.int32)])
                hv = plsc.load_gather(hist, [jnp.full((_L,), b, jnp.int32)])
                st = sv[0]
                en = st + hv[0]
                is_tail = (lo + b) == (_NBLK - 1)
                bp = jnp.full((_L,), ph, jnp.int32)

                def item(it, cnt2):
                    e = plsc.load_gather(olist, [jnp.full((_L,), it, jnp.int32)])[0]
                    pos = e & 16383
                    rs = jnp.full((_L,), lax.shift_right_logical(e, 14), jnp.int32)
                    slot = cnt2 & 7

                    @pl.when(cnt2 >= 8)
                    def _():
                        pltpu.make_async_copy(
                            table_hbm.at[0, pl.ds(0, _D)], ring.at[slot], so
                        ).wait()

                    @pl.when(jnp.logical_not(is_tail))
                    def _():
                        for c in range(_D // _L):
                            dims = iota + c * _L
                            ring[slot, pl.ds(c * _L, _L)] = plsc.load_gather(
                                blk, [bp, dims, rs]
                            )

                    @pl.when(is_tail)
                    def _():
                        for c in range(_D // _L):
                            dims = iota + c * _L
                            ring[slot, pl.ds(c * _L, _L)] = plsc.load_gather(
                                tbuf, [dims, rs]
                            )

                    pltpu.async_copy(
                        ring.at[slot],
                        out_hbm.at[pl.ds(pl.multiple_of(pos * _D, 8), _D)],
                        so,
                    )
                    return cnt2 + 1

                return lax.fori_loop(st, en, item, cnt)

            cnt = lax.fori_loop(0, nn, stream, jnp.int32(0))
            for j in range(8):
                @pl.when(j < cnt)
                def _():
                    pltpu.make_async_copy(
                        table_hbm.at[0, pl.ds(0, _D)], ring.at[j], so
                    ).wait()

        phase(u_hbm, embT_hbm, xg_hbm)
        phase(v_hbm, ctxT_hbm, yg_hbm)

    return k(u, v, embT, ctxT)


def _dot_sc(xg, yg):
    mesh = plsc.VectorSubcoreMesh(core_axis_name="c", subcore_axis_name="s")

    @functools.partial(
        pl.kernel,
        mesh=mesh,
        compiler_params=_SC_PARAMS,
        out_type=jax.ShapeDtypeStruct((_B,), jnp.float32),
        scratch_types=[
            pltpu.VMEM((_BPW * _D,), jnp.float32),
            pltpu.VMEM((_BPW * _D,), jnp.float32),
            pltpu.VMEM((_BPW,), jnp.float32),
            pltpu.VMEM((_L * _L,), jnp.float32),
        ],
    )
    def k(xg_hbm, yg_hbm, out_hbm, xv, yv, dv, tb):
        wid = lax.axis_index("s") * _NC + lax.axis_index("c")
        base = wid * _BPW
        pltpu.sync_copy(xg_hbm.at[pl.ds(base * _D, _BPW * _D)], xv)
        pltpu.sync_copy(yg_hbm.at[pl.ds(base * _D, _BPW * _D)], yv)
        iota = lax.iota(jnp.int32, _L)
        tcols = iota * _L

        def grp(g, carry):
            for j in range(_L):
                off = (g * _L + j) * _D
                t = xv[pl.ds(off, _L)] * yv[pl.ds(off, _L)]
                for c in range(1, _D // _L):
                    t = t + xv[pl.ds(off + c * _L, _L)] * yv[pl.ds(off + c * _L, _L)]
                plsc.store_scatter(tb, [tcols + j], t)
            accv = tb[pl.ds(0, _L)]
            for t in range(1, _L):
                accv = accv + tb[pl.ds(t * _L, _L)]
            dv[pl.ds(pl.multiple_of(g * _L, _L), _L)] = accv
            return carry

        lax.fori_loop(0, _BPW // _L, grp, 0)
        pltpu.sync_copy(dv, out_hbm.at[pl.ds(base, _BPW)])

    return k(xg, yg)


def _loss_tc(d, w):
    def k(d_ref, w_ref, o_ref):
        s = w_ref[...] * d_ref[...]
        ls = jnp.minimum(s, 0.0) - jnp.log1p(jnp.exp(-jnp.abs(s)))
        o_ref[0, 0] = -jnp.sum(ls) * (1.0 / _B)

    out = pl.pallas_call(
        k,
        out_shape=jax.ShapeDtypeStruct((1, 1), jnp.float32),
        out_specs=pl.BlockSpec(memory_space=pltpu.SMEM),
    )(d.reshape(128, 128), w.reshape(128, 128))
    return out[0, 0]


@jax.jit
def kernel(u, v, w, emb, ctx):
    u = u.astype(jnp.int32)
    v = v.astype(jnp.int32)
    xg, yg = _gather_sc(u, v, emb.T, ctx.T)
    d = _dot_sc(xg, yg)
    return _loss_tc(d, w.astype(jnp.float32))


# speedup vs baseline: 3.8410x; 1.2681x over previous
import functools

import jax
import jax.numpy as jnp
from jax import lax
from jax.experimental import pallas as pl
from jax.experimental.pallas import tpu as pltpu
from jax.experimental.pallas import tpu_sc as plsc

_N = 1000000
_B = 16384
_D = 64
_NC = 2
_NS = 16
_L = 16
_NW = _NC * _NS
_BPW = _B // _NW
_NBLK = (_N + 127) // 128
_TAIL = (_N // 128) * 128
_BASE = _NBLK // _NW
_REM = _NBLK % _NW
_MAXBLK = _BASE + 1

_SC_PARAMS = pltpu.CompilerParams(
    needs_layout_passes=False,
    skip_device_barrier=True,
    disable_bounds_checks=True,
    disable_semaphore_checks=True,
)


def _gather_sc(u, v, embT, ctxT):
    mesh = plsc.VectorSubcoreMesh(core_axis_name="c", subcore_axis_name="s")

    @functools.partial(
        pl.kernel,
        mesh=mesh,
        compiler_params=_SC_PARAMS,
        out_type=(
            jax.ShapeDtypeStruct((_B * _D,), jnp.float32),
            jax.ShapeDtypeStruct((_B * _D,), jnp.float32),
        ),
        scratch_types=[
            pltpu.VMEM((_B,), jnp.int32),
            pltpu.VMEM((_B,), jnp.int32),
            pltpu.VMEM((_B,), jnp.int32),
            pltpu.VMEM((256,), jnp.int32),
            pltpu.VMEM((256,), jnp.int32),
            pltpu.VMEM((256,), jnp.int32),
            pltpu.VMEM((256,), jnp.int32),
            pltpu.VMEM((3, _D, 128), jnp.float32),
            pltpu.VMEM((_D, _N - _TAIL), jnp.float32),
            pltpu.VMEM((8, _D), jnp.float32),
            pltpu.SemaphoreType.DMA,
            pltpu.SemaphoreType.DMA,
        ],
    )
    def k(
        u_hbm, v_hbm, embT_hbm, ctxT_hbm, xg_hbm, yg_hbm,
        idxs, plist, olist, hist, starts, wrk, nlist, blk, tbuf, ring, sb, so,
    ):
        wid = lax.axis_index("s") * _NC + lax.axis_index("c")
        lo = wid * _BASE + jnp.minimum(wid, _REM)
        nblk = _BASE + (wid < _REM).astype(jnp.int32)

        iota = lax.iota(jnp.int32, _L)
        ones = jnp.ones((_L,), jnp.int32)

        def fetch(table_hbm, b, ph):
            gb = lo + b

            @pl.when(gb < _NBLK - 1)
            def _():
                pltpu.async_copy(
                    table_hbm.at[:, pl.ds(pl.multiple_of(gb * 128, 128), 128)],
                    blk.at[ph],
                    sb,
                )

            @pl.when(gb == _NBLK - 1)
            def _():
                pltpu.async_copy(
                    table_hbm.at[:, pl.ds(_TAIL, _N - _TAIL)], tbuf, sb
                )

        def drain(table_hbm, b, ph):
            gb = lo + b

            @pl.when(gb < _NBLK - 1)
            def _():
                pltpu.make_async_copy(
                    table_hbm.at[:, pl.ds(0, 128)], blk.at[ph], sb
                ).wait()

            @pl.when(gb == _NBLK - 1)
            def _():
                pltpu.make_async_copy(
                    table_hbm.at[:, pl.ds(_TAIL, _N - _TAIL)], tbuf, sb
                ).wait()

        def phase(idx_hbm, table_hbm, out_hbm):
            pltpu.sync_copy(idx_hbm, idxs)
            for kk in range(256 // _L):
                hist[pl.ds(kk * _L, _L)] = jnp.zeros((_L,), jnp.int32)

            def match(kk, off):
                vals = idxs[pl.ds(pl.multiple_of(kk * _L, _L), _L)]
                blkv = lax.shift_right_logical(vals, 7)
                m = (blkv >= lo) & (blkv < lo + nblk)
                pos = kk * _L + iota
                plsc.store_compressed(plist.at[pl.ds(off, _L)], pos, mask=m)
                brel = jnp.where(m, blkv - lo, 0)
                for j in range(_L):
                    mj = m & (iota == j)
                    plsc.addupdate_scatter(hist, [brel], ones, mask=mj)
                cnt = plsc.all_reduce_population_count(m)
                return off + cnt[0]

            nm = lax.fori_loop(0, _B // _L, match, jnp.int32(0))

            def scan(kk, carry):
                c = hist[pl.ds(pl.multiple_of(kk * _L, _L), _L)]
                cum = plsc.cumsum(c)
                ex = cum - c + carry
                starts[pl.ds(pl.multiple_of(kk * _L, _L), _L)] = ex
                wrk[pl.ds(pl.multiple_of(kk * _L, _L), _L)] = ex
                return carry + cum[_L - 1]

            lax.fori_loop(0, 256 // _L, scan, jnp.int32(0))

            def place(kk, carry):
                pos16 = plist[pl.ds(pl.multiple_of(kk * _L, _L), _L)]
                valid = (kk * _L + iota) < nm
                pos16 = jnp.where(valid, pos16, 0)
                vals = plsc.load_gather(idxs, [pos16])
                brel = jnp.where(
                    valid, lax.shift_right_logical(vals, 7) - lo, 0
                )
                packed = pos16 | lax.shift_left(vals & 127, 14)
                for j in range(_L):
                    mj = valid & (iota == j)
                    slotv = plsc.load_gather(wrk, [brel])
                    plsc.store_scatter(olist, [slotv], packed, mask=mj)
                    plsc.addupdate_scatter(wrk, [brel], ones, mask=mj)
                return carry

            lax.fori_loop(0, (nm + _L - 1) // _L, place, jnp.int32(0))

            def needed(kk, offn):
                ids = kk * _L + iota
                m = hist[pl.ds(pl.multiple_of(kk * _L, _L), _L)] > 0
                plsc.store_compressed(nlist.at[pl.ds(offn, _L)], ids, mask=m)
                cn = plsc.all_reduce_population_count(m)
                return offn + cn[0]

            nn = lax.fori_loop(0, 256 // _L, needed, jnp.int32(0))

            def blk_at(i):
                return plsc.load_gather(nlist, [jnp.full((_L,), i, jnp.int32)])[0]

            @pl.when(nn > 0)
            def _():
                fetch(table_hbm, blk_at(jnp.int32(0)), 0)

            @pl.when(nn > 1)
            def _():
                fetch(table_hbm, blk_at(jnp.int32(1)), 1)

            def stream(i, cnt):
                b = blk_at(i)
                ph = lax.rem(i, 3)

                @pl.when(i + 2 < nn)
                def _():
                    fetch(table_hbm, blk_at(i + 2), lax.rem(i + 2, 3))

                drain(table_hbm, b, ph)
                sv = plsc.load_gather(starts, [jnp.full((_L,), b, jnp.int32)])
                hv = plsc.load_gather(hist, [jnp.full((_L,), b, jnp.int32)])
                st = sv[0]
                en = st + hv[0]
                is_tail = (lo + b) == (_NBLK - 1)
                bp = jnp.full((_L,), ph, jnp.int32)

                def item(it, cnt2):
                    e = plsc.load_gather(olist, [jnp.full((_L,), it, jnp.int32)])[0]
                    pos = e & 16383
                    rs = jnp.full((_L,), lax.shift_right_logical(e, 14), jnp.int32)
                    slot = cnt2 & 7

                    @pl.when(cnt2 >= 8)
                    def _():
                        pltpu.make_async_copy(
                            table_hbm.at[0, pl.ds(0, _D)], ring.at[slot], so
                        ).wait()

                    @pl.when(jnp.logical_not(is_tail))
                    def _():
                        for c in range(_D // _L):
                            dims = iota + c * _L
                            ring[slot, pl.ds(c * _L, _L)] = plsc.load_gather(
                                blk, [bp, dims, rs]
                            )

                    @pl.when(is_tail)
                    def _():
                        for c in range(_D // _L):
                            dims = iota + c * _L
                            ring[slot, pl.ds(c * _L, _L)] = plsc.load_gather(
                                tbuf, [dims, rs]
                            )

                    pltpu.async_copy(
                        ring.at[slot],
                        out_hbm.at[pl.ds(pl.multiple_of(pos * _D, 8), _D)],
                        so,
                    )
                    return cnt2 + 1

                return lax.fori_loop(st, en, item, cnt)

            cnt = lax.fori_loop(0, nn, stream, jnp.int32(0))
            for j in range(8):
                @pl.when(j < cnt)
                def _():
                    pltpu.make_async_copy(
                        table_hbm.at[0, pl.ds(0, _D)], ring.at[j], so
                    ).wait()

        phase(u_hbm, embT_hbm, xg_hbm)
        phase(v_hbm, ctxT_hbm, yg_hbm)

    return k(u, v, embT, ctxT)


def _dot_sc(xg, yg):
    mesh = plsc.VectorSubcoreMesh(core_axis_name="c", subcore_axis_name="s")

    @functools.partial(
        pl.kernel,
        mesh=mesh,
        compiler_params=_SC_PARAMS,
        out_type=jax.ShapeDtypeStruct((_B,), jnp.float32),
        scratch_types=[
            pltpu.VMEM((_BPW * _D,), jnp.float32),
            pltpu.VMEM((_BPW * _D,), jnp.float32),
            pltpu.VMEM((_BPW,), jnp.float32),
            pltpu.VMEM((_L * _L,), jnp.float32),
        ],
    )
    def k(xg_hbm, yg_hbm, out_hbm, xv, yv, dv, tb):
        wid = lax.axis_index("s") * _NC + lax.axis_index("c")
        base = wid * _BPW
        pltpu.sync_copy(xg_hbm.at[pl.ds(base * _D, _BPW * _D)], xv)
        pltpu.sync_copy(yg_hbm.at[pl.ds(base * _D, _BPW * _D)], yv)
        iota = lax.iota(jnp.int32, _L)
        tcols = iota * _L

        def grp(g, carry):
            for j in range(_L):
                off = (g * _L + j) * _D
                t = xv[pl.ds(off, _L)] * yv[pl.ds(off, _L)]
                for c in range(1, _D // _L):
                    t = t + xv[pl.ds(off + c * _L, _L)] * yv[pl.ds(off + c * _L, _L)]
                plsc.store_scatter(tb, [tcols + j], t)
            accv = tb[pl.ds(0, _L)]
            for t in range(1, _L):
                accv = accv + tb[pl.ds(t * _L, _L)]
            dv[pl.ds(pl.multiple_of(g * _L, _L), _L)] = accv
            return carry

        lax.fori_loop(0, _BPW // _L, grp, 0)
        pltpu.sync_copy(dv, out_hbm.at[pl.ds(base, _BPW)])

    return k(xg, yg)


def _loss_tc(d, w):
    def k(d_ref, w_ref, o_ref):
        s = w_ref[...] * d_ref[...]
        ls = jnp.minimum(s, 0.0) - jnp.log1p(jnp.exp(-jnp.abs(s)))
        o_ref[0, 0] = -jnp.sum(ls) * (1.0 / _B)

    out = pl.pallas_call(
        k,
        out_shape=jax.ShapeDtypeStruct((1, 1), jnp.float32),
        out_specs=pl.BlockSpec(memory_space=pltpu.SMEM),
    )(d.reshape(128, 128), w.reshape(128, 128))
    return out[0, 0]


@jax.jit
def kernel(u, v, w, emb, ctx):
    u = u.astype(jnp.int32)
    v = v.astype(jnp.int32)
    xg, yg = _gather_sc(u, v, emb.T, ctx.T)
    d = _dot_sc(xg, yg)
    return _loss_tc(d, w.astype(jnp.float32))


# speedup vs baseline: 4.3901x; 1.1430x over previous
import functools

import jax
import jax.numpy as jnp
from jax import lax
from jax.experimental import pallas as pl
from jax.experimental.pallas import tpu as pltpu
from jax.experimental.pallas import tpu_sc as plsc

_N = 1000000
_B = 16384
_D = 64
_NC = 2
_NS = 16
_L = 16
_NW = _NC * _NS
_BPW = _B // _NW
_NBLK = (_N + 127) // 128
_TAIL = (_N // 128) * 128
_BASE = _NBLK // _NW
_REM = _NBLK % _NW
_MAXBLK = _BASE + 1

_SC_PARAMS = pltpu.CompilerParams(
    needs_layout_passes=False,
    skip_device_barrier=True,
    disable_bounds_checks=True,
    disable_semaphore_checks=True,
)


def _gather_sc(u, v, embT, ctxT):
    mesh = plsc.VectorSubcoreMesh(core_axis_name="c", subcore_axis_name="s")

    @functools.partial(
        pl.kernel,
        mesh=mesh,
        compiler_params=_SC_PARAMS,
        out_type=(
            jax.ShapeDtypeStruct((_B * _D,), jnp.float32),
            jax.ShapeDtypeStruct((_B * _D,), jnp.float32),
        ),
        scratch_types=[
            pltpu.VMEM((_B,), jnp.int32),
            pltpu.VMEM((_B,), jnp.int32),
            pltpu.VMEM((_B,), jnp.int32),
            pltpu.VMEM((256,), jnp.int32),
            pltpu.VMEM((256,), jnp.int32),
            pltpu.VMEM((256,), jnp.int32),
            pltpu.VMEM((256,), jnp.int32),
            pltpu.VMEM((4, _D, 128), jnp.float32),
            pltpu.VMEM((_D, _N - _TAIL), jnp.float32),
            pltpu.VMEM((8, _D), jnp.float32),
            pltpu.SemaphoreType.DMA,
            pltpu.SemaphoreType.DMA,
        ],
    )
    def k(
        u_hbm, v_hbm, embT_hbm, ctxT_hbm, xg_hbm, yg_hbm,
        idxs, plist, olist, hist, starts, wrk, nlist, blk, tbuf, ring, sb, so,
    ):
        wid = lax.axis_index("s") * _NC + lax.axis_index("c")
        lo = wid * _BASE + jnp.minimum(wid, _REM)
        nblk = _BASE + (wid < _REM).astype(jnp.int32)

        iota = lax.iota(jnp.int32, _L)
        ones = jnp.ones((_L,), jnp.int32)

        def fetch(table_hbm, b, ph):
            gb = lo + b

            @pl.when(gb < _NBLK - 1)
            def _():
                pltpu.async_copy(
                    table_hbm.at[:, pl.ds(pl.multiple_of(gb * 128, 128), 128)],
                    blk.at[ph],
                    sb,
                )

            @pl.when(gb == _NBLK - 1)
            def _():
                pltpu.async_copy(
                    table_hbm.at[:, pl.ds(_TAIL, _N - _TAIL)], tbuf, sb
                )

        def drain(table_hbm, b, ph):
            gb = lo + b

            @pl.when(gb < _NBLK - 1)
            def _():
                pltpu.make_async_copy(
                    table_hbm.at[:, pl.ds(0, 128)], blk.at[ph], sb
                ).wait()

            @pl.when(gb == _NBLK - 1)
            def _():
                pltpu.make_async_copy(
                    table_hbm.at[:, pl.ds(_TAIL, _N - _TAIL)], tbuf, sb
                ).wait()

        def phase(idx_hbm, table_hbm, out_hbm):
            pltpu.sync_copy(idx_hbm, idxs)
            for kk in range(256 // _L):
                hist[pl.ds(kk * _L, _L)] = jnp.zeros((_L,), jnp.int32)

            def match(kk, off):
                vals = idxs[pl.ds(pl.multiple_of(kk * _L, _L), _L)]
                blkv = lax.shift_right_logical(vals, 7)
                m = (blkv >= lo) & (blkv < lo + nblk)
                pos = kk * _L + iota
                plsc.store_compressed(plist.at[pl.ds(off, _L)], pos, mask=m)
                brel = jnp.where(m, blkv - lo, 0)
                for j in range(_L):
                    mj = m & (iota == j)
                    plsc.addupdate_scatter(hist, [brel], ones, mask=mj)
                cnt = plsc.all_reduce_population_count(m)
                return off + cnt[0]

            nm = lax.fori_loop(0, _B // _L, match, jnp.int32(0))

            def scan(kk, carry):
                c = hist[pl.ds(pl.multiple_of(kk * _L, _L), _L)]
                cum = plsc.cumsum(c)
                ex = cum - c + carry
                starts[pl.ds(pl.multiple_of(kk * _L, _L), _L)] = ex
                wrk[pl.ds(pl.multiple_of(kk * _L, _L), _L)] = ex
                return carry + cum[_L - 1]

            lax.fori_loop(0, 256 // _L, scan, jnp.int32(0))

            def place(kk, carry):
                pos16 = plist[pl.ds(pl.multiple_of(kk * _L, _L), _L)]
                valid = (kk * _L + iota) < nm
                pos16 = jnp.where(valid, pos16, 0)
                vals = plsc.load_gather(idxs, [pos16])
                brel = jnp.where(
                    valid, lax.shift_right_logical(vals, 7) - lo, 0
                )
                packed = pos16 | lax.shift_left(vals & 127, 14)
                for j in range(_L):
                    mj = valid & (iota == j)
                    slotv = plsc.load_gather(wrk, [brel])
                    plsc.store_scatter(olist, [slotv], packed, mask=mj)
                    plsc.addupdate_scatter(wrk, [brel], ones, mask=mj)
                return carry

            lax.fori_loop(0, (nm + _L - 1) // _L, place, jnp.int32(0))

            def needed(kk, offn):
                ids = kk * _L + iota
                m = hist[pl.ds(pl.multiple_of(kk * _L, _L), _L)] > 0
                plsc.store_compressed(nlist.at[pl.ds(offn, _L)], ids, mask=m)
                cn = plsc.all_reduce_population_count(m)
                return offn + cn[0]

            nn = lax.fori_loop(0, 256 // _L, needed, jnp.int32(0))

            def blk_at(i):
                return plsc.load_gather(nlist, [jnp.full((_L,), i, jnp.int32)])[0]

            @pl.when(nn > 0)
            def _():
                fetch(table_hbm, blk_at(jnp.int32(0)), 0)

            @pl.when(nn > 1)
            def _():
                fetch(table_hbm, blk_at(jnp.int32(1)), 1)

            @pl.when(nn > 2)
            def _():
                fetch(table_hbm, blk_at(jnp.int32(2)), 2)

            def stream(i, cnt):
                b = blk_at(i)
                ph = i & 3

                @pl.when(i + 3 < nn)
                def _():
                    fetch(table_hbm, blk_at(i + 3), (i + 3) & 3)

                drain(table_hbm, b, ph)
                sv = plsc.load_gather(starts, [jnp.full((_L,), b, jnp.int32)])
                hv = plsc.load_gather(hist, [jnp.full((_L,), b, jnp.int32)])
                st = sv[0]
                en = st + hv[0]
                is_tail = (lo + b) == (_NBLK - 1)
                bp = jnp.full((_L,), ph, jnp.int32)

                def item(it, cnt2):
                    e = plsc.load_gather(olist, [jnp.full((_L,), it, jnp.int32)])[0]
                    pos = e & 16383
                    rs = jnp.full((_L,), lax.shift_right_logical(e, 14), jnp.int32)
                    slot = cnt2 & 7

                    @pl.when(cnt2 >= 8)
                    def _():
                        pltpu.make_async_copy(
                            table_hbm.at[0, pl.ds(0, _D)], ring.at[slot], so
                        ).wait()

                    @pl.when(jnp.logical_not(is_tail))
                    def _():
                        for c in range(_D // _L):
                            dims = iota + c * _L
                            ring[slot, pl.ds(c * _L, _L)] = plsc.load_gather(
                                blk, [bp, dims, rs]
                            )

                    @pl.when(is_tail)
                    def _():
                        for c in range(_D // _L):
                            dims = iota + c * _L
                            ring[slot, pl.ds(c * _L, _L)] = plsc.load_gather(
                                tbuf, [dims, rs]
                            )

                    pltpu.async_copy(
                        ring.at[slot],
                        out_hbm.at[pl.ds(pl.multiple_of(pos * _D, 8), _D)],
                        so,
                    )
                    return cnt2 + 1

                return lax.fori_loop(st, en, item, cnt)

            cnt = lax.fori_loop(0, nn, stream, jnp.int32(0))
            for j in range(8):
                @pl.when(j < cnt)
                def _():
                    pltpu.make_async_copy(
                        table_hbm.at[0, pl.ds(0, _D)], ring.at[j], so
                    ).wait()

        phase(u_hbm, embT_hbm, xg_hbm)
        phase(v_hbm, ctxT_hbm, yg_hbm)

    return k(u, v, embT, ctxT)


def _dot_sc(xg, yg):
    mesh = plsc.VectorSubcoreMesh(core_axis_name="c", subcore_axis_name="s")

    @functools.partial(
        pl.kernel,
        mesh=mesh,
        compiler_params=_SC_PARAMS,
        out_type=jax.ShapeDtypeStruct((_B,), jnp.float32),
        scratch_types=[
            pltpu.VMEM((_BPW * _D,), jnp.float32),
            pltpu.VMEM((_BPW * _D,), jnp.float32),
            pltpu.VMEM((_BPW,), jnp.float32),
            pltpu.VMEM((_L * _L,), jnp.float32),
        ],
    )
    def k(xg_hbm, yg_hbm, out_hbm, xv, yv, dv, tb):
        wid = lax.axis_index("s") * _NC + lax.axis_index("c")
        base = wid * _BPW
        pltpu.sync_copy(xg_hbm.at[pl.ds(base * _D, _BPW * _D)], xv)
        pltpu.sync_copy(yg_hbm.at[pl.ds(base * _D, _BPW * _D)], yv)
        iota = lax.iota(jnp.int32, _L)
        tcols = iota * _L

        def grp(g, carry):
            for j in range(_L):
                off = (g * _L + j) * _D
                t = xv[pl.ds(off, _L)] * yv[pl.ds(off, _L)]
                for c in range(1, _D // _L):
                    t = t + xv[pl.ds(off + c * _L, _L)] * yv[pl.ds(off + c * _L, _L)]
                plsc.store_scatter(tb, [tcols + j], t)
            accv = tb[pl.ds(0, _L)]
            for t in range(1, _L):
                accv = accv + tb[pl.ds(t * _L, _L)]
            dv[pl.ds(pl.multiple_of(g * _L, _L), _L)] = accv
            return carry

        lax.fori_loop(0, _BPW // _L, grp, 0)
        pltpu.sync_copy(dv, out_hbm.at[pl.ds(base, _BPW)])

    return k(xg, yg)


def _loss_tc(d, w):
    def k(d_ref, w_ref, o_ref):
        s = w_ref[...] * d_ref[...]
        ls = jnp.minimum(s, 0.0) - jnp.log1p(jnp.exp(-jnp.abs(s)))
        o_ref[0, 0] = -jnp.sum(ls) * (1.0 / _B)

    out = pl.pallas_call(
        k,
        out_shape=jax.ShapeDtypeStruct((1, 1), jnp.float32),
        out_specs=pl.BlockSpec(memory_space=pltpu.SMEM),
    )(d.reshape(128, 128), w.reshape(128, 128))
    return out[0, 0]


@jax.jit
def kernel(u, v, w, emb, ctx):
    u = u.astype(jnp.int32)
    v = v.astype(jnp.int32)
    xg, yg = _gather_sc(u, v, emb.T, ctx.T)
    d = _dot_sc(xg, yg)
    return _loss_tc(d, w.astype(jnp.float32))


# speedup vs baseline: 4.7792x; 1.0886x over previous
import functools

import jax
import jax.numpy as jnp
from jax import lax
from jax.experimental import pallas as pl
from jax.experimental.pallas import tpu as pltpu
from jax.experimental.pallas import tpu_sc as plsc

_N = 1000000
_B = 16384
_D = 64
_NC = 2
_NS = 16
_L = 16
_NW = _NC * _NS
_BPW = _B // _NW
_NBLK = (_N + 127) // 128
_TAIL = (_N // 128) * 128
_BASE = _NBLK // _NW
_REM = _NBLK % _NW
_MAXBLK = _BASE + 1

_SC_PARAMS = pltpu.CompilerParams(
    needs_layout_passes=False,
    skip_device_barrier=True,
    disable_bounds_checks=True,
    disable_semaphore_checks=True,
)


def _gather_sc(u, v, embT, ctxT):
    mesh = plsc.VectorSubcoreMesh(core_axis_name="c", subcore_axis_name="s")

    @functools.partial(
        pl.kernel,
        mesh=mesh,
        compiler_params=_SC_PARAMS,
        out_type=(
            jax.ShapeDtypeStruct((_B * _D,), jnp.float32),
            jax.ShapeDtypeStruct((_B * _D,), jnp.float32),
        ),
        scratch_types=[
            pltpu.VMEM((_B,), jnp.int32),
            pltpu.VMEM((_B,), jnp.int32),
            pltpu.VMEM((_B,), jnp.int32),
            pltpu.VMEM((256,), jnp.int32),
            pltpu.VMEM((256,), jnp.int32),
            pltpu.VMEM((256,), jnp.int32),
            pltpu.VMEM((256,), jnp.int32),
            pltpu.VMEM((6, _D, 128), jnp.float32),
            pltpu.VMEM((_D, _N - _TAIL), jnp.float32),
            pltpu.VMEM((8, _D), jnp.float32),
            pltpu.SemaphoreType.DMA,
            pltpu.SemaphoreType.DMA,
        ],
    )
    def k(
        u_hbm, v_hbm, embT_hbm, ctxT_hbm, xg_hbm, yg_hbm,
        idxs, plist, olist, hist, starts, wrk, nlist, blk, tbuf, ring, sb, so,
    ):
        wid = lax.axis_index("s") * _NC + lax.axis_index("c")
        lo = wid * _BASE + jnp.minimum(wid, _REM)
        nblk = _BASE + (wid < _REM).astype(jnp.int32)

        iota = lax.iota(jnp.int32, _L)
        ones = jnp.ones((_L,), jnp.int32)

        def fetch(table_hbm, b, ph):
            gb = lo + b

            @pl.when(gb < _NBLK - 1)
            def _():
                pltpu.async_copy(
                    table_hbm.at[:, pl.ds(pl.multiple_of(gb * 128, 128), 128)],
                    blk.at[ph],
                    sb,
                )

            @pl.when(gb == _NBLK - 1)
            def _():
                pltpu.async_copy(
                    table_hbm.at[:, pl.ds(_TAIL, _N - _TAIL)], tbuf, sb
                )

        def drain(table_hbm, b, ph):
            gb = lo + b

            @pl.when(gb < _NBLK - 1)
            def _():
                pltpu.make_async_copy(
                    table_hbm.at[:, pl.ds(0, 128)], blk.at[ph], sb
                ).wait()

            @pl.when(gb == _NBLK - 1)
            def _():
                pltpu.make_async_copy(
                    table_hbm.at[:, pl.ds(_TAIL, _N - _TAIL)], tbuf, sb
                ).wait()

        def phase(idx_hbm, table_hbm, out_hbm):
            pltpu.sync_copy(idx_hbm, idxs)
            for kk in range(256 // _L):
                hist[pl.ds(kk * _L, _L)] = jnp.zeros((_L,), jnp.int32)

            def match(kk, off):
                vals = idxs[pl.ds(pl.multiple_of(kk * _L, _L), _L)]
                blkv = lax.shift_right_logical(vals, 7)
                m = (blkv >= lo) & (blkv < lo + nblk)
                pos = kk * _L + iota
                plsc.store_compressed(plist.at[pl.ds(off, _L)], pos, mask=m)
                brel = jnp.where(m, blkv - lo, 0)
                for j in range(_L):
                    mj = m & (iota == j)
                    plsc.addupdate_scatter(hist, [brel], ones, mask=mj)
                cnt = plsc.all_reduce_population_count(m)
                return off + cnt[0]

            nm = lax.fori_loop(0, _B // _L, match, jnp.int32(0))

            def scan(kk, carry):
                c = hist[pl.ds(pl.multiple_of(kk * _L, _L), _L)]
                cum = plsc.cumsum(c)
                ex = cum - c + carry
                starts[pl.ds(pl.multiple_of(kk * _L, _L), _L)] = ex
                wrk[pl.ds(pl.multiple_of(kk * _L, _L), _L)] = ex
                return carry + cum[_L - 1]

            lax.fori_loop(0, 256 // _L, scan, jnp.int32(0))

            def place(kk, carry):
                pos16 = plist[pl.ds(pl.multiple_of(kk * _L, _L), _L)]
                valid = (kk * _L + iota) < nm
                pos16 = jnp.where(valid, pos16, 0)
                vals = plsc.load_gather(idxs, [pos16])
                brel = jnp.where(
                    valid, lax.shift_right_logical(vals, 7) - lo, 0
                )
                packed = pos16 | lax.shift_left(vals & 127, 14)
                for j in range(_L):
                    mj = valid & (iota == j)
                    slotv = plsc.load_gather(wrk, [brel])
                    plsc.store_scatter(olist, [slotv], packed, mask=mj)
                    plsc.addupdate_scatter(wrk, [brel], ones, mask=mj)
                return carry

            lax.fori_loop(0, (nm + _L - 1) // _L, place, jnp.int32(0))

            def needed(kk, offn):
                ids = kk * _L + iota
                m = hist[pl.ds(pl.multiple_of(kk * _L, _L), _L)] > 0
                plsc.store_compressed(nlist.at[pl.ds(offn, _L)], ids, mask=m)
                cn = plsc.all_reduce_population_count(m)
                return offn + cn[0]

            nn = lax.fori_loop(0, 256 // _L, needed, jnp.int32(0))

            def blk_at(i):
                return plsc.load_gather(nlist, [jnp.full((_L,), i, jnp.int32)])[0]

            @pl.when(nn > 0)
            def _():
                fetch(table_hbm, blk_at(jnp.int32(0)), 0)

            @pl.when(nn > 1)
            def _():
                fetch(table_hbm, blk_at(jnp.int32(1)), 1)

            for pre in range(2, 5):
                @pl.when(nn > pre)
                def _():
                    fetch(table_hbm, blk_at(jnp.int32(pre)), pre)

            def stream(i, cnt):
                b = blk_at(i)
                ph = lax.rem(i, 6)

                @pl.when(i + 5 < nn)
                def _():
                    fetch(table_hbm, blk_at(i + 5), lax.rem(i + 5, 6))

                drain(table_hbm, b, ph)
                sv = plsc.load_gather(starts, [jnp.full((_L,), b, jnp.int32)])
                hv = plsc.load_gather(hist, [jnp.full((_L,), b, jnp.int32)])
                st = sv[0]
                en = st + hv[0]
                is_tail = (lo + b) == (_NBLK - 1)
                bp = jnp.full((_L,), ph, jnp.int32)

                def item(it, cnt2):
                    e = plsc.load_gather(olist, [jnp.full((_L,), it, jnp.int32)])[0]
                    pos = e & 16383
                    rs = jnp.full((_L,), lax.shift_right_logical(e, 14), jnp.int32)
                    slot = cnt2 & 7

                    @pl.when(cnt2 >= 8)
                    def _():
                        pltpu.make_async_copy(
                            table_hbm.at[0, pl.ds(0, _D)], ring.at[slot], so
                        ).wait()

                    @pl.when(jnp.logical_not(is_tail))
                    def _():
                        for c in range(_D // _L):
                            dims = iota + c * _L
                            ring[slot, pl.ds(c * _L, _L)] = plsc.load_gather(
                                blk, [bp, dims, rs]
                            )

                    @pl.when(is_tail)
                    def _():
                        for c in range(_D // _L):
                            dims = iota + c * _L
                            ring[slot, pl.ds(c * _L, _L)] = plsc.load_gather(
                                tbuf, [dims, rs]
                            )

                    pltpu.async_copy(
                        ring.at[slot],
                        out_hbm.at[pl.ds(pl.multiple_of(pos * _D, 8), _D)],
                        so,
                    )
                    return cnt2 + 1

                return lax.fori_loop(st, en, item, cnt)

            cnt = lax.fori_loop(0, nn, stream, jnp.int32(0))
            for j in range(8):
                @pl.when(j < cnt)
                def _():
                    pltpu.make_async_copy(
                        table_hbm.at[0, pl.ds(0, _D)], ring.at[j], so
                    ).wait()

        phase(u_hbm, embT_hbm, xg_hbm)
        phase(v_hbm, ctxT_hbm, yg_hbm)

    return k(u, v, embT, ctxT)


def _dot_sc(xg, yg):
    mesh = plsc.VectorSubcoreMesh(core_axis_name="c", subcore_axis_name="s")

    @functools.partial(
        pl.kernel,
        mesh=mesh,
        compiler_params=_SC_PARAMS,
        out_type=jax.ShapeDtypeStruct((_B,), jnp.float32),
        scratch_types=[
            pltpu.VMEM((_BPW * _D,), jnp.float32),
            pltpu.VMEM((_BPW * _D,), jnp.float32),
            pltpu.VMEM((_BPW,), jnp.float32),
            pltpu.VMEM((_L * _L,), jnp.float32),
        ],
    )
    def k(xg_hbm, yg_hbm, out_hbm, xv, yv, dv, tb):
        wid = lax.axis_index("s") * _NC + lax.axis_index("c")
        base = wid * _BPW
        pltpu.sync_copy(xg_hbm.at[pl.ds(base * _D, _BPW * _D)], xv)
        pltpu.sync_copy(yg_hbm.at[pl.ds(base * _D, _BPW * _D)], yv)
        iota = lax.iota(jnp.int32, _L)
        tcols = iota * _L

        def grp(g, carry):
            for j in range(_L):
                off = (g * _L + j) * _D
                t = xv[pl.ds(off, _L)] * yv[pl.ds(off, _L)]
                for c in range(1, _D // _L):
                    t = t + xv[pl.ds(off + c * _L, _L)] * yv[pl.ds(off + c * _L, _L)]
                plsc.store_scatter(tb, [tcols + j], t)
            accv = tb[pl.ds(0, _L)]
            for t in range(1, _L):
                accv = accv + tb[pl.ds(t * _L, _L)]
            dv[pl.ds(pl.multiple_of(g * _L, _L), _L)] = accv
            return carry

        lax.fori_loop(0, _BPW // _L, grp, 0)
        pltpu.sync_copy(dv, out_hbm.at[pl.ds(base, _BPW)])

    return k(xg, yg)


def _loss_tc(d, w):
    def k(d_ref, w_ref, o_ref):
        s = w_ref[...] * d_ref[...]
        ls = jnp.minimum(s, 0.0) - jnp.log1p(jnp.exp(-jnp.abs(s)))
        o_ref[0, 0] = -jnp.sum(ls) * (1.0 / _B)

    out = pl.pallas_call(
        k,
        out_shape=jax.ShapeDtypeStruct((1, 1), jnp.float32),
        out_specs=pl.BlockSpec(memory_space=pltpu.SMEM),
    )(d.reshape(128, 128), w.reshape(128, 128))
    return out[0, 0]


@jax.jit
def kernel(u, v, w, emb, ctx):
    u = u.astype(jnp.int32)
    v = v.astype(jnp.int32)
    xg, yg = _gather_sc(u, v, emb.T, ctx.T)
    d = _dot_sc(xg, yg)
    return _loss_tc(d, w.astype(jnp.float32))
